# baseline (speedup 1.0000x reference)
import functools

import jax
import jax.numpy as jnp
from jax import lax
from jax.experimental import pallas as pl
from jax.experimental.pallas import tpu as pltpu
from jax.experimental.pallas import tpu_sc as plsc

N_GC = 10000
N_GN = 10000
E = 320000
H = 128

NC = 2
NS = 16
NW = NC * NS
L = 16


_NODE_BLK = 1000


def _node_proj_body(nf_gc, nf_gn, Wgc1, bgc1, Wgn1, bgn1, Wrgc1, brgc1,
                    Wrgc2, brgc2, Wrgn1, brgn1, Wrgn2, brgn2, Wgn2, bgn2,
                    h_gc_o, h_gn_o, out_gn_o):
    h_gc = jnp.dot(nf_gc[...], Wgc1[...], preferred_element_type=jnp.float32) + bgc1[...]
    h_gc = (jnp.dot(jnp.dot(h_gc, Wrgc1[...], preferred_element_type=jnp.float32) + brgc1[...],
                    Wrgc2[...], preferred_element_type=jnp.float32) + brgc2[...]) + h_gc
    h_gn = jnp.dot(nf_gn[...], Wgn1[...], preferred_element_type=jnp.float32) + bgn1[...]
    h_gn = (jnp.dot(jnp.dot(h_gn, Wrgn1[...], preferred_element_type=jnp.float32) + brgn1[...],
                    Wrgn2[...], preferred_element_type=jnp.float32) + brgn2[...]) + h_gn
    h_gc_o[...] = h_gc
    h_gn_o[...] = h_gn
    out_gn_o[...] = jnp.dot(h_gn, Wgn2[...], preferred_element_type=jnp.float32) + bgn2[...]


def _node_proj(nf_gc, nf_gn, Wgc1, bgc1, Wgn1, bgn1, Wrgc1, brgc1, Wrgc2,
               brgc2, Wrgn1, brgn1, Wrgn2, brgn2, Wgn2, bgn2):
    grid = (N_GC // _NODE_BLK,)
    row_spec = pl.BlockSpec((_NODE_BLK, H), lambda i: (i, 0))
    w_spec = pl.BlockSpec((H, H), lambda i: (0, 0))
    b_spec = pl.BlockSpec((1, H), lambda i: (0, 0))
    return pl.pallas_call(
        _node_proj_body,
        grid=grid,
        in_specs=[row_spec, row_spec] + [w_spec, b_spec] * 7,
        out_specs=[row_spec, row_spec, row_spec],
        out_shape=[jax.ShapeDtypeStruct((N_GC, H), jnp.float32)] * 3,
    )(nf_gc, nf_gn, Wgc1, bgc1, Wgn1, bgn1, Wrgc1, brgc1, Wrgc2, brgc2,
      Wrgn1, brgn1, Wrgn2, brgn2, Wgn2, bgn2)


_G_CHUNK = 400
_EPW = E // NW


def _gather_body(h_gn_hbm, h_gc_hbm, src_hbm, dst_hbm, g1_hbm, g2_hbm,
                 idx1_v, idx2_v, rows1_v, rows2_v, sem1, sem2):
    c = lax.axis_index("c")
    s = lax.axis_index("s")
    wid = s * NC + c
    base = wid * _EPW

    def step(i, carry):
        off = base + i * _G_CHUNK
        pltpu.sync_copy(src_hbm.at[pl.ds(off, _G_CHUNK)], idx1_v)
        pltpu.sync_copy(dst_hbm.at[pl.ds(off, _G_CHUNK)], idx2_v)
        cp1 = pltpu.async_copy(h_gn_hbm.at[idx1_v], rows1_v, sem1)
        cp2 = pltpu.async_copy(h_gc_hbm.at[idx2_v], rows2_v, sem2)
        cp1.wait()
        cp2.wait()
        pltpu.sync_copy(rows1_v, g1_hbm.at[pl.ds(off, _G_CHUNK)])
        pltpu.sync_copy(rows2_v, g2_hbm.at[pl.ds(off, _G_CHUNK)])
        return carry

    lax.fori_loop(0, _EPW // _G_CHUNK, step, 0)


def _edge_gather(h_gn, h_gc, src, dst):
    mesh = plsc.VectorSubcoreMesh(core_axis_name="c", subcore_axis_name="s")
    f = functools.partial(
        pl.kernel,
        out_type=[jax.ShapeDtypeStruct((E, H), jnp.float32)] * 2,
        mesh=mesh,
        scratch_types=[
            pltpu.VMEM((_G_CHUNK,), jnp.int32),
            pltpu.VMEM((_G_CHUNK,), jnp.int32),
            pltpu.VMEM((_G_CHUNK, H), jnp.float32),
            pltpu.VMEM((_G_CHUNK, H), jnp.float32),
            pltpu.SemaphoreType.DMA,
            pltpu.SemaphoreType.DMA,
        ],
    )(_gather_body)
    return f(h_gn, h_gc, src, dst)


_E_BLK = 2560


def _edge_mlp_body(g1, g2, W1a, W1b, b1, Wk, bk, Wf1, bf1, Wf2, bf2,
                   efo1_o, efo2_o):
    x = (jnp.dot(g1[...], W1a[...], preferred_element_type=jnp.float32)
         + jnp.dot(g2[...], W1b[...], preferred_element_type=jnp.float32)
         + b1[...])
    x = jnp.where(x >= 0.0, x, 0.2 * x)
    k = jax.nn.sigmoid(jnp.dot(x, Wk[...], preferred_element_type=jnp.float32) + bk[...])
    kcol = k[:, 0:1]
    f1 = jnp.dot(x, Wf1[...], preferred_element_type=jnp.float32) + bf1[...]
    f2 = jnp.dot(x, Wf2[...], preferred_element_type=jnp.float32) + bf2[...]
    efo1_o[...] = f1 * kcol
    efo2_o[...] = f2 * kcol


def _edge_mlp(g1, g2, W1a, W1b, b1, Wk, bk, Wf1, bf1, Wf2, bf2):
    grid = (E // _E_BLK,)
    e_spec = pl.BlockSpec((_E_BLK, H), lambda i: (i, 0))
    return pl.pallas_call(
        _edge_mlp_body,
        grid=grid,
        in_specs=[
            e_spec, e_spec,
            pl.BlockSpec((H, 2 * H), lambda i: (0, 0)),
            pl.BlockSpec((H, 2 * H), lambda i: (0, 0)),
            pl.BlockSpec((1, 2 * H), lambda i: (0, 0)),
            pl.BlockSpec((2 * H, 8), lambda i: (0, 0)),
            pl.BlockSpec((1, 8), lambda i: (0, 0)),
            pl.BlockSpec((2 * H, H), lambda i: (0, 0)),
            pl.BlockSpec((1, H), lambda i: (0, 0)),
            pl.BlockSpec((2 * H, H), lambda i: (0, 0)),
            pl.BlockSpec((1, H), lambda i: (0, 0)),
        ],
        out_specs=[e_spec, e_spec],
        out_shape=[jax.ShapeDtypeStruct((E, H), jnp.float32)] * 2,
    )(g1, g2, W1a, W1b, b1, Wk, bk, Wf1, bf1, Wf2, bf2)


_S_CHUNK = 200
_ACC_ROWS = 10240
_RPS = _ACC_ROWS // NS


def _seg_sum_body(dst_hbm, efo1_hbm, sum_hbm, acc_sh, dstv, rows1, zrow):
    c = lax.axis_index("c")
    s = lax.axis_index("s")
    wid = s * NC + c

    def zstep(i, carry):
        for j in range(H // L):
            zrow[i, pl.ds(j * L, L)] = jnp.zeros((L,), jnp.float32)
        return carry
    lax.fori_loop(0, 128, zstep, 0)
    for r in range(_RPS // 128):
        pltpu.sync_copy(zrow, acc_sh.at[pl.ds(s * _RPS + r * 128, 128)])
    plsc.subcore_barrier()

    base = wid * _EPW

    def sum_step(i, carry):
        off = base + i * _S_CHUNK
        pltpu.sync_copy(dst_hbm.at[pl.ds(off, _S_CHUNK)], dstv)
        pltpu.sync_copy(efo1_hbm.at[pl.ds(off, _S_CHUNK)], rows1)
        pltpu.sync_copy(rows1, acc_sh.at[dstv], add=True)
        return carry
    lax.fori_loop(0, _EPW // _S_CHUNK, sum_step, 0)

    plsc.subcore_barrier()
    pltpu.sync_copy(acc_sh.at[pl.ds(s * _RPS, _RPS)],
                    sum_hbm.at[c, pl.ds(s * _RPS, _RPS)])


def _seg_sum(dst, efo1):
    mesh = plsc.VectorSubcoreMesh(core_axis_name="c", subcore_axis_name="s")
    f = functools.partial(
        pl.kernel,
        out_type=jax.ShapeDtypeStruct((NC, _ACC_ROWS, H), jnp.float32),
        mesh=mesh,
        scratch_types=[
            pltpu.VMEM_SHARED((_ACC_ROWS, H), jnp.float32),
            pltpu.VMEM((_S_CHUNK,), jnp.int32),
            pltpu.VMEM((_S_CHUNK, H), jnp.float32),
            pltpu.VMEM((128, H), jnp.float32),
        ],
        compiler_params=pltpu.CompilerParams(needs_layout_passes=False),
    )(_seg_sum_body)
    return f(dst, efo1)


_M_STAGE = 12800
_M_GC = 128
_M_K = 3
_M_ROWS = 100
_NPT = 320
_NEG = -3.0e38


def _seg_max_body(dst_hbm, efo2_hbm, max_hbm,
                  stage, ids, dloc, rows2, accmax, sem2):
    c = lax.axis_index("c")
    s = lax.axis_index("s")
    wid = s * NC + c
    lo = jnp.minimum(wid * _NPT, N_GC - _NPT)
    hi = lo + _NPT

    def istep(i, carry):
        def ij(j, c2):
            accmax[i, pl.ds(j * L, L)] = jnp.full((L,), _NEG, jnp.float32)
            return c2
        lax.fori_loop(0, H // L, ij, 0)
        return carry
    lax.fori_loop(0, _NPT, istep, 0)

    def jstep(i, carry):
        def jj(j, c2):
            ids[i, pl.ds(j * L, L)] = jnp.zeros((L,), jnp.int32)
            return c2
        lax.fori_loop(0, _M_GC // L, jj, 0)
        return carry
    lax.fori_loop(0, _M_ROWS, jstep, 0)

    def outer(o, carry):
        ebase = o * _M_STAGE
        pltpu.sync_copy(dst_hbm.at[pl.ds(ebase, _M_STAGE)], stage)

        def filt(i, cnt):
            d = stage[pl.ds(i * L, L)]
            m = (d >= lo) & (d < hi)
            eid = ebase + i * L + lax.iota(jnp.int32, L)
            mi = m.astype(jnp.int32)
            lane = lax.iota(jnp.int32, L)
            dnums = lax.GatherDimensionNumbers(
                offset_dims=(), collapsed_slice_dims=(0,), start_index_map=(0,))
            v = mi
            for k in (1, 2, 4, 8):
                idx = jnp.maximum(lane - k, 0)
                g = lax.gather(v, idx[:, None], dnums, (1,),
                               mode=lax.GatherScatterMode.PROMISE_IN_BOUNDS)
                v = v + jnp.where(lane >= k, g, 0)
            pos = cnt + v - 1
            row = lax.shift_right_logical(pos, 7)
            col = pos & (_M_GC - 1)
            plsc.store_scatter(ids, [row, col], eid, mask=m)
            plsc.store_scatter(dloc, [pos], d - lo, mask=m)
            return cnt + plsc.all_reduce_population_count(m)[0]
        cnt = lax.fori_loop(0, _M_STAGE // L, filt, jnp.int32(0))

        def group(q, carry):
            gbase = q * (_M_K * _M_GC)
            for b in range(_M_K):
                @pl.when(gbase + b * _M_GC < cnt)
                def _fire(b=b):
                    pltpu.async_copy(
                        efo2_hbm.at[ids.at[q * _M_K + b]],
                        rows2.at[pl.ds(b * _M_GC, _M_GC)], sem2)
            for b in range(_M_K):
                @pl.when(gbase + b * _M_GC < cnt)
                def _drain(b=b):
                    pltpu.make_async_copy(
                        efo2_hbm.at[ids.at[q * _M_K + b]],
                        rows2.at[pl.ds(b * _M_GC, _M_GC)], sem2).wait()

            n_here = jnp.minimum(cnt - gbase, _M_K * _M_GC)

            def red(i, carry2):
                dl = dloc[pl.ds(gbase + i, L)][0]
                def rj(j, c2):
                    sl = pl.ds(j * L, L)
                    accmax[dl, sl] = jnp.maximum(accmax[dl, sl], rows2[i, sl])
                    return c2
                lax.fori_loop(0, H // L, rj, 0)
                return carry2
            lax.fori_loop(0, n_here, red, 0)
            return carry
        lax.fori_loop(0, (_M_ROWS + _M_K - 1) // _M_K, group, 0)
        return carry
    lax.fori_loop(0, E // _M_STAGE, outer, 0)

    pltpu.sync_copy(accmax, max_hbm.at[pl.ds(lo, _NPT)])


def _seg_max(dst, efo2):
    mesh = plsc.VectorSubcoreMesh(core_axis_name="c", subcore_axis_name="s")
    f = functools.partial(
        pl.kernel,
        out_type=jax.ShapeDtypeStruct((N_GC, H), jnp.float32),
        mesh=mesh,
        scratch_types=[
            pltpu.VMEM((_M_STAGE,), jnp.int32),
            pltpu.VMEM((_M_ROWS, _M_GC), jnp.int32),
            pltpu.VMEM((_M_STAGE + L,), jnp.int32),
            pltpu.VMEM((_M_K * _M_GC, H), jnp.float32),
            pltpu.VMEM((_NPT, H), jnp.float32),
            pltpu.SemaphoreType.DMA,
        ],
        compiler_params=pltpu.CompilerParams(needs_layout_passes=False),
    )(_seg_max_body)
    return f(dst, efo2)


def _final_body(h_gc, sum0, sum1, max_raw, Wra, Wrb, Wrc, bred, W2a, W2b,
                bgc2, out_o):
    nfo1 = sum0[0] + sum1[0]
    mr = max_raw[...]
    nfo2 = jnp.where(mr <= -1.0e38, 0.0, mr)
    hg = h_gc[...]
    new_x = (jnp.dot(hg, Wra[...], preferred_element_type=jnp.float32)
             + jnp.dot(nfo1, Wrb[...], preferred_element_type=jnp.float32)
             + jnp.dot(nfo2, Wrc[...], preferred_element_type=jnp.float32)
             + bred[...])
    out_o[...] = (jnp.dot(hg, W2a[...], preferred_element_type=jnp.float32)
                  + jnp.dot(new_x, W2b[...], preferred_element_type=jnp.float32)
                  + bgc2[...])


def _final(h_gc, sum_p, max_raw, Wra, Wrb, Wrc, bred, W2a, W2b, bgc2):
    grid = (N_GC // _NODE_BLK,)
    row_spec = pl.BlockSpec((_NODE_BLK, H), lambda i: (i, 0))
    sum_spec0 = pl.BlockSpec((1, _NODE_BLK, H), lambda i: (0, i, 0))
    sum_spec1 = pl.BlockSpec((1, _NODE_BLK, H), lambda i: (1, i, 0))
    w_spec = pl.BlockSpec((H, H), lambda i: (0, 0))
    b_spec = pl.BlockSpec((1, H), lambda i: (0, 0))
    return pl.pallas_call(
        _final_body,
        grid=grid,
        in_specs=[row_spec, sum_spec0, sum_spec1, row_spec,
                  w_spec, w_spec, w_spec, b_spec, w_spec, w_spec, b_spec],
        out_specs=pl.BlockSpec((_NODE_BLK, H), lambda i: (i, 0)),
        out_shape=jax.ShapeDtypeStruct((N_GC, H), jnp.float32),
    )(h_gc, sum_p, sum_p, max_raw, Wra, Wrb, Wrc, bred, W2a, W2b, bgc2)


def kernel(nf_gc, nf_gn, edge_index,
           W_gc1, b_gc1, W_gn1, b_gn1,
           W_rgc1, b_rgc1, W_rgc2, b_rgc2,
           W_rgn1, b_rgn1, W_rgn2, b_rgn2,
           W_msg1, b_msg1, W_msg2, b_msg2,
           W_red, b_red, W_gc2, b_gc2, W_gn2, b_gn2):
    src = edge_index[0].astype(jnp.int32)
    dst = edge_index[1].astype(jnp.int32)

    r2 = lambda b: b.reshape(1, -1)

    h_gc, h_gn, out_gn = _node_proj(
        nf_gc, nf_gn, W_gc1.T, r2(b_gc1), W_gn1.T, r2(b_gn1),
        W_rgc1.T, r2(b_rgc1), W_rgc2.T, r2(b_rgc2),
        W_rgn1.T, r2(b_rgn1), W_rgn2.T, r2(b_rgn2),
        W_gn2.T, r2(b_gn2))

    g1, g2 = _edge_gather(h_gn, h_gc, src, dst)

    W1t = W_msg1.T
    W2t = W_msg2.T
    Wk = jnp.pad(W2t[:, 0:1], ((0, 0), (0, 7)))
    bk = jnp.pad(b_msg2[0:1], (0, 7)).reshape(1, 8)
    efo1, efo2 = _edge_mlp(
        g1, g2, W1t[:H], W1t[H:], r2(b_msg1), Wk, bk,
        W2t[:, 1:1 + H], r2(b_msg2[1:1 + H]),
        W2t[:, 1 + H:], r2(b_msg2[1 + H:]))

    sum_p = _seg_sum(dst, efo1)
    max_raw = _seg_max(dst, efo2)

    Wrt = W_red.T
    W2ct = W_gc2.T
    out_gc = _final(h_gc, sum_p, max_raw,
                    Wrt[:H], Wrt[H:2 * H], Wrt[2 * H:], r2(b_red),
                    W2ct[:H], W2ct[H:], r2(b_gc2))
    return (out_gc, out_gn)


# speedup vs baseline: 1.2781x; 1.2781x over previous
import functools

import jax
import jax.numpy as jnp
from jax import lax
from jax.experimental import pallas as pl
from jax.experimental.pallas import tpu as pltpu
from jax.experimental.pallas import tpu_sc as plsc

N_GC = 10000
N_GN = 10000
E = 320000
H = 128

NC = 2
NS = 16
NW = NC * NS
L = 16


_NODE_BLK = 1000


def _node_proj_body(nf_gc, nf_gn, Wgc1, bgc1, Wgn1, bgn1, Wrgc1, brgc1,
                    Wrgc2, brgc2, Wrgn1, brgn1, Wrgn2, brgn2, Wgn2, bgn2,
                    h_gc_o, h_gn_o, out_gn_o):
    h_gc = jnp.dot(nf_gc[...], Wgc1[...], preferred_element_type=jnp.float32) + bgc1[...]
    h_gc = (jnp.dot(jnp.dot(h_gc, Wrgc1[...], preferred_element_type=jnp.float32) + brgc1[...],
                    Wrgc2[...], preferred_element_type=jnp.float32) + brgc2[...]) + h_gc
    h_gn = jnp.dot(nf_gn[...], Wgn1[...], preferred_element_type=jnp.float32) + bgn1[...]
    h_gn = (jnp.dot(jnp.dot(h_gn, Wrgn1[...], preferred_element_type=jnp.float32) + brgn1[...],
                    Wrgn2[...], preferred_element_type=jnp.float32) + brgn2[...]) + h_gn
    h_gc_o[...] = h_gc
    h_gn_o[...] = h_gn
    out_gn_o[...] = jnp.dot(h_gn, Wgn2[...], preferred_element_type=jnp.float32) + bgn2[...]


def _node_proj(nf_gc, nf_gn, Wgc1, bgc1, Wgn1, bgn1, Wrgc1, brgc1, Wrgc2,
               brgc2, Wrgn1, brgn1, Wrgn2, brgn2, Wgn2, bgn2):
    grid = (N_GC // _NODE_BLK,)
    row_spec = pl.BlockSpec((_NODE_BLK, H), lambda i: (i, 0))
    w_spec = pl.BlockSpec((H, H), lambda i: (0, 0))
    b_spec = pl.BlockSpec((1, H), lambda i: (0, 0))
    return pl.pallas_call(
        _node_proj_body,
        grid=grid,
        in_specs=[row_spec, row_spec] + [w_spec, b_spec] * 7,
        out_specs=[row_spec, row_spec, row_spec],
        out_shape=[jax.ShapeDtypeStruct((N_GC, H), jnp.float32)] * 3,
    )(nf_gc, nf_gn, Wgc1, bgc1, Wgn1, bgn1, Wrgc1, brgc1, Wrgc2, brgc2,
      Wrgn1, brgn1, Wrgn2, brgn2, Wgn2, bgn2)


_G_CHUNK = 400
_EPW = E // NW


def _gather_body(h_gn_hbm, h_gc_hbm, src_hbm, dst_hbm, g1_hbm, g2_hbm,
                 idx1_v, idx2_v, rows1_v, rows2_v, sem1, sem2):
    c = lax.axis_index("c")
    s = lax.axis_index("s")
    wid = s * NC + c
    base = wid * _EPW

    def step(i, carry):
        off = base + i * _G_CHUNK
        pltpu.sync_copy(src_hbm.at[pl.ds(off, _G_CHUNK)], idx1_v)
        pltpu.sync_copy(dst_hbm.at[pl.ds(off, _G_CHUNK)], idx2_v)
        cp1 = pltpu.async_copy(h_gn_hbm.at[idx1_v], rows1_v, sem1)
        cp2 = pltpu.async_copy(h_gc_hbm.at[idx2_v], rows2_v, sem2)
        cp1.wait()
        cp2.wait()
        pltpu.sync_copy(rows1_v, g1_hbm.at[pl.ds(off, _G_CHUNK)])
        pltpu.sync_copy(rows2_v, g2_hbm.at[pl.ds(off, _G_CHUNK)])
        return carry

    lax.fori_loop(0, _EPW // _G_CHUNK, step, 0)


def _edge_gather(h_gn, h_gc, src, dst):
    mesh = plsc.VectorSubcoreMesh(core_axis_name="c", subcore_axis_name="s")
    f = functools.partial(
        pl.kernel,
        out_type=[jax.ShapeDtypeStruct((E, H), jnp.float32)] * 2,
        mesh=mesh,
        scratch_types=[
            pltpu.VMEM((_G_CHUNK,), jnp.int32),
            pltpu.VMEM((_G_CHUNK,), jnp.int32),
            pltpu.VMEM((_G_CHUNK, H), jnp.float32),
            pltpu.VMEM((_G_CHUNK, H), jnp.float32),
            pltpu.SemaphoreType.DMA,
            pltpu.SemaphoreType.DMA,
        ],
    )(_gather_body)
    return f(h_gn, h_gc, src, dst)


_E_BLK = 2560


def _edge_mlp_body(g1, g2, W1a, W1b, b1, Wk, bk, Wf1, bf1, Wf2, bf2,
                   efo1_o, efo2_o):
    x = (jnp.dot(g1[...], W1a[...], preferred_element_type=jnp.float32)
         + jnp.dot(g2[...], W1b[...], preferred_element_type=jnp.float32)
         + b1[...])
    x = jnp.where(x >= 0.0, x, 0.2 * x)
    k = jax.nn.sigmoid(jnp.dot(x, Wk[...], preferred_element_type=jnp.float32) + bk[...])
    kcol = k[:, 0:1]
    f1 = jnp.dot(x, Wf1[...], preferred_element_type=jnp.float32) + bf1[...]
    f2 = jnp.dot(x, Wf2[...], preferred_element_type=jnp.float32) + bf2[...]
    efo1_o[...] = f1 * kcol
    efo2_o[...] = f2 * kcol


def _edge_mlp(g1, g2, W1a, W1b, b1, Wk, bk, Wf1, bf1, Wf2, bf2):
    grid = (E // _E_BLK,)
    e_spec = pl.BlockSpec((_E_BLK, H), lambda i: (i, 0))
    return pl.pallas_call(
        _edge_mlp_body,
        grid=grid,
        in_specs=[
            e_spec, e_spec,
            pl.BlockSpec((H, 2 * H), lambda i: (0, 0)),
            pl.BlockSpec((H, 2 * H), lambda i: (0, 0)),
            pl.BlockSpec((1, 2 * H), lambda i: (0, 0)),
            pl.BlockSpec((2 * H, 8), lambda i: (0, 0)),
            pl.BlockSpec((1, 8), lambda i: (0, 0)),
            pl.BlockSpec((2 * H, H), lambda i: (0, 0)),
            pl.BlockSpec((1, H), lambda i: (0, 0)),
            pl.BlockSpec((2 * H, H), lambda i: (0, 0)),
            pl.BlockSpec((1, H), lambda i: (0, 0)),
        ],
        out_specs=[e_spec, e_spec],
        out_shape=[jax.ShapeDtypeStruct((E, H), jnp.float32)] * 2,
    )(g1, g2, W1a, W1b, b1, Wk, bk, Wf1, bf1, Wf2, bf2)


_S_CHUNK = 200
_ACC_ROWS = 10240
_RPS = _ACC_ROWS // NS


def _seg_sum_body(dst_hbm, efo1_hbm, sum_hbm, acc_sh, dstv, rows1, zrow):
    c = lax.axis_index("c")
    s = lax.axis_index("s")
    wid = s * NC + c

    def zstep(i, carry):
        for j in range(H // L):
            zrow[i, pl.ds(j * L, L)] = jnp.zeros((L,), jnp.float32)
        return carry
    lax.fori_loop(0, 128, zstep, 0)
    for r in range(_RPS // 128):
        pltpu.sync_copy(zrow, acc_sh.at[pl.ds(s * _RPS + r * 128, 128)])
    plsc.subcore_barrier()

    base = wid * _EPW

    def sum_step(i, carry):
        off = base + i * _S_CHUNK
        pltpu.sync_copy(dst_hbm.at[pl.ds(off, _S_CHUNK)], dstv)
        pltpu.sync_copy(efo1_hbm.at[pl.ds(off, _S_CHUNK)], rows1)
        pltpu.sync_copy(rows1, acc_sh.at[dstv], add=True)
        return carry
    lax.fori_loop(0, _EPW // _S_CHUNK, sum_step, 0)

    plsc.subcore_barrier()
    pltpu.sync_copy(acc_sh.at[pl.ds(s * _RPS, _RPS)],
                    sum_hbm.at[c, pl.ds(s * _RPS, _RPS)])


def _seg_sum(dst, efo1):
    mesh = plsc.VectorSubcoreMesh(core_axis_name="c", subcore_axis_name="s")
    f = functools.partial(
        pl.kernel,
        out_type=jax.ShapeDtypeStruct((NC, _ACC_ROWS, H), jnp.float32),
        mesh=mesh,
        scratch_types=[
            pltpu.VMEM_SHARED((_ACC_ROWS, H), jnp.float32),
            pltpu.VMEM((_S_CHUNK,), jnp.int32),
            pltpu.VMEM((_S_CHUNK, H), jnp.float32),
            pltpu.VMEM((128, H), jnp.float32),
        ],
        compiler_params=pltpu.CompilerParams(needs_layout_passes=False),
    )(_seg_sum_body)
    return f(dst, efo1)


_M_STAGE = 10000
_M_GC = 128
_M_K = 3
_M_ROWS = 79
_NPT = 320
_NEG = -3.0e38


def _seg_max_body(dst_hbm, efo2_hbm, max_hbm,
                  stage, ids, dloc, rows2, accmax, sem2):
    c = lax.axis_index("c")
    s = lax.axis_index("s")
    wid = s * NC + c
    lo = jnp.minimum(wid * _NPT, N_GC - _NPT)
    hi = lo + _NPT

    def istep(i, carry):
        def ij(j, c2):
            accmax[i, pl.ds(j * L, L)] = jnp.full((L,), _NEG, jnp.float32)
            return c2
        lax.fori_loop(0, H // L, ij, 0)
        return carry
    lax.fori_loop(0, _NPT, istep, 0)

    def jstep(i, carry):
        def jj(j, c2):
            ids[i, pl.ds(j * L, L)] = jnp.zeros((L,), jnp.int32)
            return c2
        lax.fori_loop(0, _M_GC // L, jj, 0)
        return carry
    lax.fori_loop(0, _M_ROWS, jstep, 0)

    def outer(o, carry):
        ebase = o * _M_STAGE
        pltpu.sync_copy(dst_hbm.at[pl.ds(ebase, _M_STAGE)], stage)

        def filt(i, cnt):
            d = stage[pl.ds(i * L, L)]
            m = (d >= lo) & (d < hi)
            eid = ebase + i * L + lax.iota(jnp.int32, L)
            mi = m.astype(jnp.int32)
            lane = lax.iota(jnp.int32, L)
            dnums = lax.GatherDimensionNumbers(
                offset_dims=(), collapsed_slice_dims=(0,), start_index_map=(0,))
            v = mi
            for k in (1, 2, 4, 8):
                idx = jnp.maximum(lane - k, 0)
                g = lax.gather(v, idx[:, None], dnums, (1,),
                               mode=lax.GatherScatterMode.PROMISE_IN_BOUNDS)
                v = v + jnp.where(lane >= k, g, 0)
            pos = cnt + v - 1
            row = lax.shift_right_logical(pos, 7)
            col = pos & (_M_GC - 1)
            plsc.store_scatter(ids, [row, col], eid, mask=m)
            plsc.store_scatter(dloc, [pos], d - lo, mask=m)
            return cnt + plsc.all_reduce_population_count(m)[0]
        cnt = lax.fori_loop(0, _M_STAGE // L, filt, jnp.int32(0))

        def group(q, carry):
            gbase = q * (_M_K * _M_GC)
            for b in range(_M_K):
                @pl.when(gbase + b * _M_GC < cnt)
                def _fire(b=b):
                    pltpu.async_copy(
                        efo2_hbm.at[ids.at[q * _M_K + b]],
                        rows2.at[pl.ds(b * _M_GC, _M_GC)], sem2)
            for b in range(_M_K):
                @pl.when(gbase + b * _M_GC < cnt)
                def _drain(b=b):
                    pltpu.make_async_copy(
                        efo2_hbm.at[ids.at[q * _M_K + b]],
                        rows2.at[pl.ds(b * _M_GC, _M_GC)], sem2).wait()

            n_here = jnp.minimum(cnt - gbase, _M_K * _M_GC)

            def red(i, carry2):
                dl = dloc[pl.ds(gbase + i, L)][0]
                def rj(j, c2):
                    sl = pl.ds(j * L, L)
                    accmax[dl, sl] = jnp.maximum(accmax[dl, sl], rows2[i, sl])
                    return c2
                lax.fori_loop(0, H // L, rj, 0)
                return carry2
            lax.fori_loop(0, n_here, red, 0)
            return carry
        lax.fori_loop(0, (_M_ROWS + _M_K - 1) // _M_K, group, 0)
        return carry
    lax.fori_loop(0, E // _M_STAGE, outer, 0)

    pltpu.sync_copy(accmax, max_hbm.at[pl.ds(lo, _NPT)])


def _seg_max(dst, efo2):
    mesh = plsc.VectorSubcoreMesh(core_axis_name="c", subcore_axis_name="s")
    f = functools.partial(
        pl.kernel,
        out_type=jax.ShapeDtypeStruct((N_GC, H), jnp.float32),
        mesh=mesh,
        scratch_types=[
            pltpu.VMEM((_M_STAGE,), jnp.int32),
            pltpu.VMEM((_M_ROWS, _M_GC), jnp.int32),
            pltpu.VMEM((_M_STAGE + L,), jnp.int32),
            pltpu.VMEM((_M_K * _M_GC, H), jnp.float32),
            pltpu.VMEM((_NPT, H), jnp.float32),
            pltpu.SemaphoreType.DMA,
        ],
        compiler_params=pltpu.CompilerParams(needs_layout_passes=False),
    )(_seg_max_body)
    return f(dst, efo2)


def _final_body(h_gc, sum0, sum1, max_raw, Wra, Wrb, Wrc, bred, W2a, W2b,
                bgc2, out_o):
    nfo1 = sum0[0] + sum1[0]
    mr = max_raw[...]
    nfo2 = jnp.where(mr <= -1.0e38, 0.0, mr)
    hg = h_gc[...]
    new_x = (jnp.dot(hg, Wra[...], preferred_element_type=jnp.float32)
             + jnp.dot(nfo1, Wrb[...], preferred_element_type=jnp.float32)
             + jnp.dot(nfo2, Wrc[...], preferred_element_type=jnp.float32)
             + bred[...])
    out_o[...] = (jnp.dot(hg, W2a[...], preferred_element_type=jnp.float32)
                  + jnp.dot(new_x, W2b[...], preferred_element_type=jnp.float32)
                  + bgc2[...])


def _final(h_gc, sum_p, max_raw, Wra, Wrb, Wrc, bred, W2a, W2b, bgc2):
    grid = (N_GC // _NODE_BLK,)
    row_spec = pl.BlockSpec((_NODE_BLK, H), lambda i: (i, 0))
    sum_spec0 = pl.BlockSpec((1, _NODE_BLK, H), lambda i: (0, i, 0))
    sum_spec1 = pl.BlockSpec((1, _NODE_BLK, H), lambda i: (1, i, 0))
    w_spec = pl.BlockSpec((H, H), lambda i: (0, 0))
    b_spec = pl.BlockSpec((1, H), lambda i: (0, 0))
    return pl.pallas_call(
        _final_body,
        grid=grid,
        in_specs=[row_spec, sum_spec0, sum_spec1, row_spec,
                  w_spec, w_spec, w_spec, b_spec, w_spec, w_spec, b_spec],
        out_specs=pl.BlockSpec((_NODE_BLK, H), lambda i: (i, 0)),
        out_shape=jax.ShapeDtypeStruct((N_GC, H), jnp.float32),
    )(h_gc, sum_p, sum_p, max_raw, Wra, Wrb, Wrc, bred, W2a, W2b, bgc2)


def kernel(nf_gc, nf_gn, edge_index,
           W_gc1, b_gc1, W_gn1, b_gn1,
           W_rgc1, b_rgc1, W_rgc2, b_rgc2,
           W_rgn1, b_rgn1, W_rgn2, b_rgn2,
           W_msg1, b_msg1, W_msg2, b_msg2,
           W_red, b_red, W_gc2, b_gc2, W_gn2, b_gn2):
    src = edge_index[0].astype(jnp.int32)
    dst = edge_index[1].astype(jnp.int32)

    r2 = lambda b: b.reshape(1, -1)

    h_gc, h_gn, out_gn = _node_proj(
        nf_gc, nf_gn, W_gc1.T, r2(b_gc1), W_gn1.T, r2(b_gn1),
        W_rgc1.T, r2(b_rgc1), W_rgc2.T, r2(b_rgc2),
        W_rgn1.T, r2(b_rgn1), W_rgn2.T, r2(b_rgn2),
        W_gn2.T, r2(b_gn2))

    g1, g2 = _edge_gather(h_gn, h_gc, src, dst)

    W1t = W_msg1.T
    W2t = W_msg2.T
    Wk = jnp.pad(W2t[:, 0:1], ((0, 0), (0, 7)))
    bk = jnp.pad(b_msg2[0:1], (0, 7)).reshape(1, 8)
    efo1, efo2 = _edge_mlp(
        g1, g2, W1t[:H], W1t[H:], r2(b_msg1), Wk, bk,
        W2t[:, 1:1 + H], r2(b_msg2[1:1 + H]),
        W2t[:, 1 + H:], r2(b_msg2[1 + H:]))

    sum_p = _seg_sum(dst, efo1)
    max_raw = _seg_max(dst, efo2)

    Wrt = W_red.T
    W2ct = W_gc2.T
    out_gc = _final(h_gc, sum_p, max_raw,
                    Wrt[:H], Wrt[H:2 * H], Wrt[2 * H:], r2(b_red),
                    W2ct[:H], W2ct[H:], r2(b_gc2))
    return (out_gc, out_gn)
